# baseline (speedup 1.0000x reference)
import functools

import jax
import jax.numpy as jnp
from jax import lax
from jax.experimental import pallas as pl
from jax.experimental.pallas import tpu as pltpu
from jax.experimental.pallas import tpu_sc as plsc

N_NODES = 10000
T_STEPS = 4
D_FEAT = 128
L_LAYERS = 2

NPAD = 10240
DUMMY_ROW = NPAD - 8
NC, NS = 2, 16
NW = NC * NS
CHUNK = 128
ROW_BLK = 256
GRID_ROWS = NPAD // ROW_BLK
TILE_ROWS = NPAD // NS

_PNAMES = ('W0', 'Wz', 'Wr', 'Wh', 'Uz', 'Ur', 'Uh', 'Bz', 'Br', 'Bh')


def _prep_edges(edge_index):
  src, dst = edge_index[0], edge_index[1]
  e = src.shape[0]
  per = -(-e // NW)
  nchunks = -(-per // CHUNK)
  perp = nchunks * CHUNK
  tot = NW * perp
  pad_dst = N_NODES + (jnp.arange(tot - e, dtype=jnp.int32) % 192)
  src = jnp.pad(src, (0, tot - e))
  dst = jnp.concatenate([dst, pad_dst])
  src = src.reshape(perp, NW).T
  dst = dst.reshape(perp, NW).T
  return (src.reshape(NC, NS, nchunks, CHUNK),
          dst.reshape(NC, NS, nchunks, CHUNK), nchunks)


def _deg_call(dstw, ones, zeros, nchunks):
  mesh = plsc.VectorSubcoreMesh(core_axis_name="c", subcore_axis_name="s")

  @functools.partial(
      pl.kernel, mesh=mesh,
      out_type=jax.ShapeDtypeStruct((NC, NPAD, D_FEAT), jnp.float32),
      scratch_types=[
          pltpu.VMEM((nchunks, CHUNK), jnp.int32),
          pltpu.VMEM((CHUNK, D_FEAT), jnp.float32),
          pltpu.VMEM_SHARED((NPAD, D_FEAT), jnp.float32),
      ],
  )
  def deg_kernel(dst_hbm, ones_hbm, zeros_hbm, out_hbm, dst_v, ones_v, acc):
    c = lax.axis_index("c")
    s = lax.axis_index("s")
    pltpu.sync_copy(dst_hbm.at[c, s], dst_v)
    pltpu.sync_copy(ones_hbm, ones_v)
    rows = pl.ds(s * TILE_ROWS, TILE_ROWS)
    pltpu.sync_copy(zeros_hbm.at[rows], acc.at[rows])
    plsc.subcore_barrier()

    def body(j, carry):
      pltpu.sync_copy(ones_v, acc.at[dst_v.at[j]], add=True)
      return carry

    lax.fori_loop(0, nchunks, body, 0)
    plsc.subcore_barrier()
    pltpu.sync_copy(acc.at[rows], out_hbm.at[c, rows])

  return deg_kernel(dstw, ones, zeros)


def _mp_call(hp, srcw, dstw, zeros, nchunks):
  mesh = plsc.VectorSubcoreMesh(core_axis_name="c", subcore_axis_name="s")

  @functools.partial(
      pl.kernel, mesh=mesh,
      out_type=jax.ShapeDtypeStruct((NC, T_STEPS, NPAD, D_FEAT), jnp.float32),
      scratch_types=[
          pltpu.VMEM((nchunks, CHUNK), jnp.int32),
          pltpu.VMEM((nchunks, CHUNK), jnp.int32),
          pltpu.VMEM((CHUNK, D_FEAT), jnp.float32),
          pltpu.VMEM_SHARED((NPAD, D_FEAT), jnp.float32),
          pltpu.SemaphoreType.DMA,
      ],
  )
  def mp_kernel(h0, h1, h2, h3, src_hbm, dst_hbm, zeros_hbm, out_hbm,
                src_v, dst_v, rows_v, acc, sem):
    c = lax.axis_index("c")
    s = lax.axis_index("s")
    pltpu.sync_copy(src_hbm.at[c, s], src_v)
    pltpu.sync_copy(dst_hbm.at[c, s], dst_v)
    rows = pl.ds(s * TILE_ROWS, TILE_ROWS)
    for t, h_hbm in enumerate((h0, h1, h2, h3)):
      pltpu.sync_copy(zeros_hbm.at[rows], acc.at[rows])
      plsc.subcore_barrier()

      def body(j, carry):
        pltpu.async_copy(h_hbm.at[src_v.at[j]], rows_v, sem).wait()
        pltpu.sync_copy(rows_v, acc.at[dst_v.at[j]], add=True)
        return carry

      lax.fori_loop(0, nchunks, body, 0)
      plsc.subcore_barrier()
      pltpu.sync_copy(acc.at[rows], out_hbm.at[c, t, rows])

  return mp_kernel(hp[0], hp[1], hp[2], hp[3], srcw, dstw, zeros)


def _evolve_kernel(deg_ref, *refs):
  prefs = refs[:len(_PNAMES)]
  wout, dinv_out = refs[len(_PNAMES):]
  p = dict(zip(_PNAMES, prefs))
  deg = deg_ref[0][:, 0:1] + deg_ref[1][:, 0:1] + 1.0
  dinv_out[...] = lax.rsqrt(deg)
  for l in range(L_LAYERS):
    h = p['W0'][l]
    for t in range(T_STEPS):
      wz = jnp.dot(p['Wz'][l], h, preferred_element_type=jnp.float32)
      uz = jnp.dot(p['Uz'][l], h, preferred_element_type=jnp.float32)
      z = jax.nn.sigmoid(wz + uz + p['Bz'][l])
      wr = jnp.dot(p['Wr'][l], h, preferred_element_type=jnp.float32)
      ur = jnp.dot(p['Ur'][l], h, preferred_element_type=jnp.float32)
      r = jax.nn.sigmoid(wr + ur + p['Br'][l])
      wh = jnp.dot(p['Wh'][l], h, preferred_element_type=jnp.float32)
      uh = jnp.dot(p['Uh'][l], r * h, preferred_element_type=jnp.float32)
      ht = jnp.tanh(wh + uh + p['Bh'][l])
      h = (1.0 - z) * h + z * ht
      wout[l, t] = h


def _evolve_call(degp, stacked):
  return pl.pallas_call(
      _evolve_kernel,
      out_shape=[
          jax.ShapeDtypeStruct((L_LAYERS, T_STEPS, D_FEAT, D_FEAT), jnp.float32),
          jax.ShapeDtypeStruct((NPAD, 1), jnp.float32),
      ],
  )(degp, *[stacked[n] for n in _PNAMES])


_DINV_SPEC = pl.BlockSpec((ROW_BLK, 1), lambda t, i: (i, 0))


def _mm0_kernel(x_ref, w_ref, dinv_ref, out_ref):
  h = jnp.dot(x_ref[...], w_ref[0], preferred_element_type=jnp.float32)
  out_ref[0] = dinv_ref[...] * h


def _mm0_call(xf, w, dinv):
  return pl.pallas_call(
      _mm0_kernel,
      grid=(T_STEPS, GRID_ROWS),
      in_specs=[
          pl.BlockSpec((ROW_BLK, D_FEAT), lambda t, i: (i, t)),
          pl.BlockSpec((1, D_FEAT, D_FEAT), lambda t, i: (t, 0, 0)),
          _DINV_SPEC,
      ],
      out_specs=pl.BlockSpec((1, ROW_BLK, D_FEAT), lambda t, i: (t, i, 0)),
      out_shape=jax.ShapeDtypeStruct((T_STEPS, NPAD, D_FEAT), jnp.float32),
  )(xf, w, dinv)


def _mm1_kernel(p_ref, h_ref, w_ref, dinv_ref, out_ref):
  dinv = dinv_ref[...]
  u = jnp.maximum(dinv * (p_ref[0, 0] + p_ref[1, 0] + h_ref[0]), 0.0)
  out_ref[0] = dinv * jnp.dot(u, w_ref[0], preferred_element_type=jnp.float32)


def _mm1_call(partials, hp, w, dinv):
  return pl.pallas_call(
      _mm1_kernel,
      grid=(T_STEPS, GRID_ROWS),
      in_specs=[
          pl.BlockSpec((NC, 1, ROW_BLK, D_FEAT), lambda t, i: (0, t, i, 0)),
          pl.BlockSpec((1, ROW_BLK, D_FEAT), lambda t, i: (t, i, 0)),
          pl.BlockSpec((1, D_FEAT, D_FEAT), lambda t, i: (t, 0, 0)),
          _DINV_SPEC,
      ],
      out_specs=pl.BlockSpec((1, ROW_BLK, D_FEAT), lambda t, i: (t, i, 0)),
      out_shape=jax.ShapeDtypeStruct((T_STEPS, NPAD, D_FEAT), jnp.float32),
  )(partials, hp, w, dinv)


def _final_kernel(p_ref, h_ref, dinv_ref, out_ref):
  dinv = dinv_ref[...]
  out_ref[...] = jnp.maximum(dinv * (p_ref[0, 0] + p_ref[1, 0] + h_ref[0]), 0.0)


def _final_call(partials, hp, dinv):
  return pl.pallas_call(
      _final_kernel,
      grid=(T_STEPS, GRID_ROWS),
      in_specs=[
          pl.BlockSpec((NC, 1, ROW_BLK, D_FEAT), lambda t, i: (0, t, i, 0)),
          pl.BlockSpec((1, ROW_BLK, D_FEAT), lambda t, i: (t, i, 0)),
          _DINV_SPEC,
      ],
      out_specs=pl.BlockSpec((ROW_BLK, D_FEAT), lambda t, i: (i, t)),
      out_shape=jax.ShapeDtypeStruct((NPAD, T_STEPS * D_FEAT), jnp.float32),
  )(partials, hp, dinv)


@jax.jit
def kernel(x, edge_index, params):
  n = x.shape[0]
  srcw, dstw, nchunks = _prep_edges(edge_index)
  xf = jnp.pad(x.reshape(n, T_STEPS * D_FEAT), ((0, NPAD - n), (0, 0)))
  zeros = jnp.zeros((NPAD, D_FEAT), jnp.float32)
  ones = jnp.ones((CHUNK, D_FEAT), jnp.float32)
  stacked = {nm: jnp.stack([p[nm] for p in params]) for nm in _PNAMES}

  degp = _deg_call(dstw, ones, zeros, nchunks)
  w_all, dinv = _evolve_call(degp, stacked)

  h1 = _mm0_call(xf, w_all[0], dinv)
  p1 = _mp_call(h1, srcw, dstw, zeros, nchunks)
  h2 = _mm1_call(p1, h1, w_all[1], dinv)
  p2 = _mp_call(h2, srcw, dstw, zeros, nchunks)
  y = _final_call(p2, h2, dinv)
  return y[:n].reshape(n, T_STEPS, D_FEAT)


# speedup vs baseline: 1.0029x; 1.0029x over previous
import functools

import jax
import jax.numpy as jnp
from jax import lax
from jax.experimental import pallas as pl
from jax.experimental.pallas import tpu as pltpu
from jax.experimental.pallas import tpu_sc as plsc

N_NODES = 10000
T_STEPS = 4
D_FEAT = 128
L_LAYERS = 2

NPAD = 10240
DUMMY_ROW = NPAD - 8
NC, NS = 2, 16
NW = NC * NS
CHUNK = 128
ROW_BLK = 256
GRID_ROWS = NPAD // ROW_BLK
TILE_ROWS = NPAD // NS

_PNAMES = ('W0', 'Wz', 'Wr', 'Wh', 'Uz', 'Ur', 'Uh', 'Bz', 'Br', 'Bh')


def _prep_edges(edge_index):
  src, dst = edge_index[0], edge_index[1]
  e = src.shape[0]
  per = -(-e // NW)
  nchunks = -(-per // CHUNK)
  perp = nchunks * CHUNK
  tot = NW * perp
  pad_dst = N_NODES + (jnp.arange(tot - e, dtype=jnp.int32) % 192)
  src = jnp.pad(src, (0, tot - e))
  dst = jnp.concatenate([dst, pad_dst])
  src = src.reshape(perp, NW).T
  dst = dst.reshape(perp, NW).T
  return (src.reshape(NC, NS, nchunks, CHUNK),
          dst.reshape(NC, NS, nchunks, CHUNK), nchunks)


def _deg_call(dstw, ones, zeros, nchunks):
  mesh = plsc.VectorSubcoreMesh(core_axis_name="c", subcore_axis_name="s")

  @functools.partial(
      pl.kernel, mesh=mesh,
      out_type=jax.ShapeDtypeStruct((NC, NPAD, D_FEAT), jnp.float32),
      scratch_types=[
          pltpu.VMEM((nchunks, CHUNK), jnp.int32),
          pltpu.VMEM((CHUNK, D_FEAT), jnp.float32),
          pltpu.VMEM_SHARED((NPAD, D_FEAT), jnp.float32),
      ],
  )
  def deg_kernel(dst_hbm, ones_hbm, zeros_hbm, out_hbm, dst_v, ones_v, acc):
    c = lax.axis_index("c")
    s = lax.axis_index("s")
    pltpu.sync_copy(dst_hbm.at[c, s], dst_v)
    pltpu.sync_copy(ones_hbm, ones_v)
    rows = pl.ds(s * TILE_ROWS, TILE_ROWS)
    pltpu.sync_copy(zeros_hbm.at[rows], acc.at[rows])
    plsc.subcore_barrier()

    def body(j, carry):
      pltpu.sync_copy(ones_v, acc.at[dst_v.at[j]], add=True)
      return carry

    lax.fori_loop(0, nchunks, body, 0)
    plsc.subcore_barrier()
    pltpu.sync_copy(acc.at[rows], out_hbm.at[c, rows])

  return deg_kernel(dstw, ones, zeros)


def _mp_call(hp, srcw, dstw, zeros, nchunks):
  mesh = plsc.VectorSubcoreMesh(core_axis_name="c", subcore_axis_name="s")

  @functools.partial(
      pl.kernel, mesh=mesh,
      out_type=jax.ShapeDtypeStruct((NC, T_STEPS, NPAD, D_FEAT), jnp.float32),
      scratch_types=[
          pltpu.VMEM((nchunks, CHUNK), jnp.int32),
          pltpu.VMEM((nchunks, CHUNK), jnp.int32),
          pltpu.VMEM((CHUNK, D_FEAT), jnp.float32),
          pltpu.VMEM_SHARED((NPAD, D_FEAT), jnp.float32),
          pltpu.SemaphoreType.DMA,
      ],
  )
  def mp_kernel(h0, h1, h2, h3, src_hbm, dst_hbm, zeros_hbm, out_hbm,
                src_v, dst_v, rows_v, acc, sem):
    c = lax.axis_index("c")
    s = lax.axis_index("s")
    pltpu.sync_copy(src_hbm.at[c, s], src_v)
    pltpu.sync_copy(dst_hbm.at[c, s], dst_v)
    rows = pl.ds(s * TILE_ROWS, TILE_ROWS)
    for t, h_hbm in enumerate((h0, h1, h2, h3)):
      pltpu.sync_copy(zeros_hbm.at[rows], acc.at[rows])
      plsc.subcore_barrier()

      def body(j, carry):
        pltpu.async_copy(h_hbm.at[src_v.at[j]], rows_v, sem).wait()
        pltpu.sync_copy(rows_v, acc.at[dst_v.at[j]], add=True)
        return carry

      lax.fori_loop(0, nchunks, body, 0)
      plsc.subcore_barrier()
      pltpu.sync_copy(acc.at[rows], out_hbm.at[c, t, rows])

  return mp_kernel(hp[0], hp[1], hp[2], hp[3], srcw, dstw, zeros)


def _evolve_kernel(deg_ref, *refs):
  prefs = refs[:len(_PNAMES)]
  wout, dinv_out = refs[len(_PNAMES):]
  p = dict(zip(_PNAMES, prefs))
  deg = deg_ref[0][:, 0:1] + deg_ref[1][:, 0:1] + 1.0
  dinv_out[...] = lax.rsqrt(deg)
  for l in range(L_LAYERS):
    h = p['W0'][l]
    for t in range(T_STEPS):
      wz = jnp.dot(p['Wz'][l], h, preferred_element_type=jnp.float32)
      uz = jnp.dot(p['Uz'][l], h, preferred_element_type=jnp.float32)
      z = jax.nn.sigmoid(wz + uz + p['Bz'][l])
      wr = jnp.dot(p['Wr'][l], h, preferred_element_type=jnp.float32)
      ur = jnp.dot(p['Ur'][l], h, preferred_element_type=jnp.float32)
      r = jax.nn.sigmoid(wr + ur + p['Br'][l])
      wh = jnp.dot(p['Wh'][l], h, preferred_element_type=jnp.float32)
      uh = jnp.dot(p['Uh'][l], r * h, preferred_element_type=jnp.float32)
      ht = jnp.tanh(wh + uh + p['Bh'][l])
      h = (1.0 - z) * h + z * ht
      wout[l, t] = h


def _evolve_call(degp, stacked):
  return pl.pallas_call(
      _evolve_kernel,
      out_shape=[
          jax.ShapeDtypeStruct((L_LAYERS, T_STEPS, D_FEAT, D_FEAT), jnp.float32),
          jax.ShapeDtypeStruct((NPAD, 1), jnp.float32),
      ],
  )(degp, *[stacked[n] for n in _PNAMES])


_DINV_SPEC = pl.BlockSpec((ROW_BLK, 1), lambda t, i: (i, 0))


def _mm0_kernel(x_ref, w_ref, dinv_ref, out_ref):
  h = jnp.dot(x_ref[0], w_ref[0], preferred_element_type=jnp.float32)
  out_ref[0] = dinv_ref[...] * h


def _mm0_call(xt, w, dinv):
  return pl.pallas_call(
      _mm0_kernel,
      grid=(T_STEPS, GRID_ROWS),
      in_specs=[
          pl.BlockSpec((1, ROW_BLK, D_FEAT), lambda t, i: (t, i, 0)),
          pl.BlockSpec((1, D_FEAT, D_FEAT), lambda t, i: (t, 0, 0)),
          _DINV_SPEC,
      ],
      out_specs=pl.BlockSpec((1, ROW_BLK, D_FEAT), lambda t, i: (t, i, 0)),
      out_shape=jax.ShapeDtypeStruct((T_STEPS, NPAD, D_FEAT), jnp.float32),
  )(xt, w, dinv)


def _mm1_kernel(p_ref, h_ref, w_ref, dinv_ref, out_ref):
  dinv = dinv_ref[...]
  u = jnp.maximum(dinv * (p_ref[0, 0] + p_ref[1, 0] + h_ref[0]), 0.0)
  out_ref[0] = dinv * jnp.dot(u, w_ref[0], preferred_element_type=jnp.float32)


def _mm1_call(partials, hp, w, dinv):
  return pl.pallas_call(
      _mm1_kernel,
      grid=(T_STEPS, GRID_ROWS),
      in_specs=[
          pl.BlockSpec((NC, 1, ROW_BLK, D_FEAT), lambda t, i: (0, t, i, 0)),
          pl.BlockSpec((1, ROW_BLK, D_FEAT), lambda t, i: (t, i, 0)),
          pl.BlockSpec((1, D_FEAT, D_FEAT), lambda t, i: (t, 0, 0)),
          _DINV_SPEC,
      ],
      out_specs=pl.BlockSpec((1, ROW_BLK, D_FEAT), lambda t, i: (t, i, 0)),
      out_shape=jax.ShapeDtypeStruct((T_STEPS, NPAD, D_FEAT), jnp.float32),
  )(partials, hp, w, dinv)


def _final_kernel(p_ref, h_ref, dinv_ref, out_ref):
  dinv = dinv_ref[...]
  out_ref[0] = jnp.maximum(dinv * (p_ref[0, 0] + p_ref[1, 0] + h_ref[0]), 0.0)


def _final_call(partials, hp, dinv):
  return pl.pallas_call(
      _final_kernel,
      grid=(T_STEPS, GRID_ROWS),
      in_specs=[
          pl.BlockSpec((NC, 1, ROW_BLK, D_FEAT), lambda t, i: (0, t, i, 0)),
          pl.BlockSpec((1, ROW_BLK, D_FEAT), lambda t, i: (t, i, 0)),
          _DINV_SPEC,
      ],
      out_specs=pl.BlockSpec((1, ROW_BLK, D_FEAT), lambda t, i: (t, i, 0)),
      out_shape=jax.ShapeDtypeStruct((T_STEPS, NPAD, D_FEAT), jnp.float32),
  )(partials, hp, dinv)


@jax.jit
def kernel(x, edge_index, params):
  n = x.shape[0]
  srcw, dstw, nchunks = _prep_edges(edge_index)
  xt = jnp.transpose(x, (1, 0, 2))
  xt = jnp.pad(xt, ((0, 0), (0, NPAD - n), (0, 0)))
  zeros = jnp.zeros((NPAD, D_FEAT), jnp.float32)
  ones = jnp.ones((CHUNK, D_FEAT), jnp.float32)
  stacked = {nm: jnp.stack([p[nm] for p in params]) for nm in _PNAMES}

  degp = _deg_call(dstw, ones, zeros, nchunks)
  w_all, dinv = _evolve_call(degp, stacked)

  h1 = _mm0_call(xt, w_all[0], dinv)
  p1 = _mp_call(h1, srcw, dstw, zeros, nchunks)
  h2 = _mm1_call(p1, h1, w_all[1], dinv)
  p2 = _mp_call(h2, srcw, dstw, zeros, nchunks)
  y = _final_call(p2, h2, dinv)
  return jnp.transpose(y[:, :n], (1, 0, 2))
